# Initial kernel scaffold; baseline (speedup 1.0000x reference)
import jax
import jax.numpy as jnp
from jax.experimental import pallas as pl


def kernel(idx, tok_table, pos_table):
    raise NotImplementedError("write your pallas kernel here")


# speedup vs baseline: 2.7438x; 2.7438x over previous
import functools

import jax
import jax.numpy as jnp
from jax import lax
from jax.experimental import pallas as pl
from jax.experimental.pallas import tpu as pltpu
from jax.experimental.pallas import tpu_sc as plsc

VOCAB = 32000
D = 256
B = 1024
T = 512
L = 16
NC = 2
NS = 16
NW = NC * NS
SEQ_PER_W = B // NW
C = 128
N_TC = T // C


def _emb_kernel(idx_hbm, tok_hbm, pos_hbm, out_hbm, idx_v, tok_v, pos_v, sem):
    wid = lax.axis_index("s") * NC + lax.axis_index("c")
    seq0 = wid * SEQ_PER_W

    for tc in range(N_TC):
        t0 = tc * C
        pltpu.sync_copy(pos_hbm.at[pl.ds(t0, C)], pos_v)

        def seq_body(s, _):
            base = (seq0 + s) * T + t0
            pltpu.sync_copy(idx_hbm.at[pl.ds(base, C)], idx_v)
            pltpu.async_copy(tok_hbm.at[idx_v], tok_v, sem).wait()

            def add_row(r, _):
                for j in range(D // L):
                    sl = pl.ds(j * L, L)
                    tok_v[r, sl] = tok_v[r, sl] + pos_v[r, sl]
                return 0

            lax.fori_loop(0, C, add_row, 0)
            pltpu.sync_copy(tok_v, out_hbm.at[pl.ds(base, C)])
            return 0

        lax.fori_loop(0, SEQ_PER_W, seq_body, 0)


@jax.jit
def kernel(idx, tok_table, pos_table):
    idx_flat = idx.reshape(B * T)
    run = pl.kernel(
        _emb_kernel,
        out_type=jax.ShapeDtypeStruct((B * T, D), jnp.float32),
        mesh=plsc.VectorSubcoreMesh(core_axis_name="c", subcore_axis_name="s"),
        scratch_types=[
            pltpu.VMEM((C,), jnp.int32),
            pltpu.VMEM((C, D), jnp.float32),
            pltpu.VMEM((C, D), jnp.float32),
            pltpu.SemaphoreType.DMA,
        ],
    )
    out = run(idx_flat, tok_table, pos_table)
    return out.reshape(B, T, D)


# speedup vs baseline: 4.9654x; 1.8096x over previous
import jax
import jax.numpy as jnp
from jax import lax
from jax.experimental import pallas as pl
from jax.experimental.pallas import tpu as pltpu
from jax.experimental.pallas import tpu_sc as plsc

VOCAB = 32000
D = 256
B = 1024
T = 512
L = 16
NC = 2
NS = 16
NW = NC * NS
SPW = B // NW
C = 64
N_TC = T // C


def _emb_kernel(idx_hbm, tok_hbm, pos_hbm, out_hbm,
                idx_v, pos_v, g0, g1, s0b, s1b,
                gsem0, gsem1, ssem0, ssem1):
    wid = lax.axis_index("s") * NC + lax.axis_index("c")
    seq0 = wid * SPW
    pltpu.sync_copy(idx_hbm.at[pl.ds(seq0 * N_TC, SPW * N_TC)], idx_v)

    def add_chunk(gbuf, sbuf):
        def row(r, _):
            for j in range(D // L):
                sl = pl.ds(j * L, L)
                sbuf[r, sl] = gbuf[r, sl] + pos_v[r, sl]
            return 0
        lax.fori_loop(0, C, row, 0)

    for tc in range(N_TC):
        t0 = tc * C

        def base(s):
            return (seq0 + s) * T + t0

        def irow(s):
            return idx_v.at[s * N_TC + tc]

        pltpu.sync_copy(pos_hbm.at[pl.ds(t0, C)], pos_v)
        pltpu.async_copy(tok_hbm.at[irow(0)], g0, gsem0)

        def pair(p, _):
            ga = 2 * p
            gb = ga + 1
            pltpu.async_copy(tok_hbm.at[irow(gb)], g1, gsem1)
            pltpu.make_async_copy(tok_hbm.at[irow(ga)], g0, gsem0).wait()

            @pl.when(p > 0)
            def _():
                pltpu.make_async_copy(
                    s0b, out_hbm.at[pl.ds(base(ga - 2), C)], ssem0).wait()

            add_chunk(g0, s0b)
            pltpu.async_copy(s0b, out_hbm.at[pl.ds(base(ga), C)], ssem0)

            @pl.when(p < SPW // 2 - 1)
            def _():
                pltpu.async_copy(tok_hbm.at[irow(ga + 2)], g0, gsem0)

            pltpu.make_async_copy(tok_hbm.at[irow(gb)], g1, gsem1).wait()

            @pl.when(p > 0)
            def _():
                pltpu.make_async_copy(
                    s1b, out_hbm.at[pl.ds(base(gb - 2), C)], ssem1).wait()

            add_chunk(g1, s1b)
            pltpu.async_copy(s1b, out_hbm.at[pl.ds(base(gb), C)], ssem1)
            return 0

        lax.fori_loop(0, SPW // 2, pair, 0)
        pltpu.make_async_copy(
            s0b, out_hbm.at[pl.ds(base(SPW - 2), C)], ssem0).wait()
        pltpu.make_async_copy(
            s1b, out_hbm.at[pl.ds(base(SPW - 1), C)], ssem1).wait()


@jax.jit
def kernel(idx, tok_table, pos_table):
    run = pl.kernel(
        _emb_kernel,
        out_type=jax.ShapeDtypeStruct((B * T, D), jnp.float32),
        mesh=plsc.VectorSubcoreMesh(core_axis_name="c", subcore_axis_name="s"),
        scratch_types=[
            pltpu.VMEM((SPW * N_TC, C), jnp.int32),
            pltpu.VMEM((C, D), jnp.float32),
            pltpu.VMEM((C, D), jnp.float32),
            pltpu.VMEM((C, D), jnp.float32),
            pltpu.VMEM((C, D), jnp.float32),
            pltpu.VMEM((C, D), jnp.float32),
            pltpu.SemaphoreType.DMA,
            pltpu.SemaphoreType.DMA,
            pltpu.SemaphoreType.DMA,
            pltpu.SemaphoreType.DMA,
        ],
    )
    out = run(idx.reshape(B * N_TC, C), tok_table, pos_table)
    return out.reshape(B, T, D)
